# Initial kernel scaffold; baseline (speedup 1.0000x reference)
import jax
import jax.numpy as jnp
from jax.experimental import pallas as pl


def kernel(ids, global_table, pos_table):
    raise NotImplementedError("write your pallas kernel here")


# speedup vs baseline: 2.4481x; 2.4481x over previous
import functools

import jax
import jax.numpy as jnp
from jax import lax
from jax.experimental import pallas as pl
from jax.experimental.pallas import tpu as pltpu
from jax.experimental.pallas import tpu_sc as plsc

_B, _T, _V, _P, _D = 4, 8192, 256, 8, 512
_L = 16
_NC, _NS = 2, 16
_NW = _NC * _NS
_TPW = _T // _NW
_C = 64
_NCHUNK = _TPW // _C


def _body(ids_hbm, table_hbm, pos_hbm, out_hbm, idx_v, pos_v, gat_v, sem):
    wid = lax.axis_index("s") * _NC + lax.axis_index("c")
    t_base = wid * _TPW

    @pl.when(wid < _B)
    def _pad():
        zero = jnp.zeros((_L,), jnp.float32)

        def zrow(r, carry):
            for j in range(_D // _L):
                gat_v[r, pl.ds(j * _L, _L)] = zero
            return carry

        lax.fori_loop(0, _P, zrow, 0)
        pltpu.sync_copy(gat_v.at[pl.ds(0, _P)], out_hbm.at[wid, pl.ds(_T, _P)])

    def chunk(ci, carry):
        t0 = t_base + ci * _C
        pltpu.sync_copy(pos_hbm.at[pl.ds(t0, _C)], pos_v)

        def batch(b, carry_b):
            pltpu.sync_copy(ids_hbm.at[b, pl.ds(t0, _C)], idx_v)
            pltpu.async_copy(table_hbm.at[idx_v], gat_v, sem).wait()

            def row(r, carry_r):
                for j in range(_D // _L):
                    sl = pl.ds(j * _L, _L)
                    gat_v[r, sl] = gat_v[r, sl] + pos_v[r, sl]
                return carry_r

            lax.fori_loop(0, _C, row, 0)
            pltpu.sync_copy(gat_v, out_hbm.at[b, pl.ds(t0, _C)])
            return carry_b

        lax.fori_loop(0, _B, batch, 0)
        return carry

    lax.fori_loop(0, _NCHUNK, chunk, 0)


_kern = functools.partial(
    pl.kernel,
    out_type=jax.ShapeDtypeStruct((_B, _T + _P, _D), jnp.float32),
    mesh=plsc.VectorSubcoreMesh(core_axis_name="c", subcore_axis_name="s"),
    scratch_types=[
        pltpu.VMEM((_C,), jnp.int32),
        pltpu.VMEM((_C, _D), jnp.float32),
        pltpu.VMEM((_C, _D), jnp.float32),
        pltpu.SemaphoreType.DMA,
    ],
)(_body)


@jax.jit
def _megabyte(ids, global_table, pos_table):
    out = _kern(ids, global_table, pos_table)
    return out.reshape(_B, _T // _P + 1, _P * _D)


def kernel(ids, global_table, pos_table):
    return _megabyte(ids, global_table, pos_table)


# speedup vs baseline: 2.7110x; 1.1074x over previous
import functools

import jax
import jax.numpy as jnp
from jax import lax
from jax.experimental import pallas as pl
from jax.experimental.pallas import tpu as pltpu
from jax.experimental.pallas import tpu_sc as plsc

_B, _T, _V, _P, _D = 4, 8192, 256, 8, 512
_L = 16
_NC, _NS = 2, 16
_NW = _NC * _NS
_TPW = _T // _NW
_C = 32
_NCHUNK = _TPW // _C
_RPB = _D // _L


def _body(ids_hbm, table_hbm, pos_hbm, out_hbm,
          idx_all, pos0, pos1, gat0, gat1,
          sg0, sg1, sw0, sw1, sp0, sp1):
    wid = lax.axis_index("s") * _NC + lax.axis_index("c")
    t_base = wid * _TPW
    gat = [gat0, gat1]
    sg = [sg0, sg1]
    sw = [sw0, sw1]
    pos = [pos0, pos1]
    sp = [sp0, sp1]

    zero = jnp.zeros((_L,), jnp.float32)

    @pl.when(wid < _B)
    def _pad():
        def zrow(r, carry):
            for j in range(_RPB):
                gat0[r, pl.ds(j * _L, _L)] = zero
            return carry

        lax.fori_loop(0, _P, zrow, 0)
        pltpu.sync_copy(gat0.at[pl.ds(0, _P)], out_hbm.at[wid, pl.ds(_T, _P)])

    pltpu.sync_copy(ids_hbm.at[:, pl.ds(t_base, _TPW)], idx_all)
    pltpu.async_copy(pos_hbm.at[pl.ds(t_base, _C)], pos0, sp0)
    pltpu.async_copy(pos_hbm.at[pl.ds(t_base + _C, _C)], pos1, sp1)
    pltpu.async_copy(table_hbm.at[idx_all.at[0, pl.ds(0, _C)]], gat0, sg0)

    def wait_write(par):
        pltpu.make_async_copy(gat[par], out_hbm.at[0, pl.ds(0, _C)], sw[par]).wait()

    def wait_gather(par):
        pltpu.make_async_copy(
            table_hbm.at[idx_all.at[0, pl.ds(0, _C)]], gat[par], sg[par]).wait()

    def wait_pos(ppar):
        pltpu.make_async_copy(pos_hbm.at[pl.ds(0, _C)], pos[ppar], sp[ppar]).wait()

    def chunk_units(c, ppar):
        t0 = t_base + c * _C
        for b in range(_B):
            par = b & 1
            npar = par ^ 1
            if b == 0:
                @pl.when(c > 0)
                def _w():
                    wait_write(npar)
            else:
                wait_write(npar)
            if b < _B - 1:
                pltpu.async_copy(
                    table_hbm.at[idx_all.at[b + 1, pl.ds(c * _C, _C)]],
                    gat[npar], sg[npar])
            else:
                @pl.when(c + 1 < _NCHUNK)
                def _g():
                    pltpu.async_copy(
                        table_hbm.at[idx_all.at[0, pl.ds((c + 1) * _C, _C)]],
                        gat[npar], sg[npar])
            if b == 0:
                wait_pos(ppar)
            wait_gather(par)
            g = gat[par]
            p = pos[ppar]

            def row(r, carry):
                for j in range(_RPB):
                    sl = pl.ds(j * _L, _L)
                    g[r, sl] = g[r, sl] + p[r, sl]
                return carry

            lax.fori_loop(0, _C, row, 0)
            pltpu.async_copy(g, out_hbm.at[b, pl.ds(t0, _C)], sw[par])
            if b == _B - 1:
                @pl.when(c + 2 < _NCHUNK)
                def _p():
                    pltpu.async_copy(
                        pos_hbm.at[pl.ds(t_base + (c + 2) * _C, _C)],
                        pos[ppar], sp[ppar])

    def cc_body(cc, carry):
        chunk_units(2 * cc, 0)
        chunk_units(2 * cc + 1, 1)
        return carry

    lax.fori_loop(0, _NCHUNK // 2, cc_body, 0)

    wait_write((_B - 1) & 1)


_kern = functools.partial(
    pl.kernel,
    out_type=jax.ShapeDtypeStruct((_B, _T + _P, _D), jnp.float32),
    mesh=plsc.VectorSubcoreMesh(core_axis_name="c", subcore_axis_name="s"),
    scratch_types=[
        pltpu.VMEM((_B, _TPW), jnp.int32),
        pltpu.VMEM((_C, _D), jnp.float32),
        pltpu.VMEM((_C, _D), jnp.float32),
        pltpu.VMEM((_C, _D), jnp.float32),
        pltpu.VMEM((_C, _D), jnp.float32),
        pltpu.SemaphoreType.DMA,
        pltpu.SemaphoreType.DMA,
        pltpu.SemaphoreType.DMA,
        pltpu.SemaphoreType.DMA,
        pltpu.SemaphoreType.DMA,
        pltpu.SemaphoreType.DMA,
    ],
)(_body)


@jax.jit
def _megabyte(ids, global_table, pos_table):
    out = _kern(ids, global_table, pos_table)
    return out.reshape(_B, _T // _P + 1, _P * _D)


def kernel(ids, global_table, pos_table):
    return _megabyte(ids, global_table, pos_table)
